# baseline (speedup 1.0000x reference)
import functools

import jax
import jax.numpy as jnp
from jax import lax
from jax.experimental import pallas as pl
from jax.experimental.pallas import tpu as pltpu
from jax.experimental.pallas import tpu_sc as plsc

N = 10000
IN_DIM = 128
OUT_DIM = 64
N_PAD = 10240
NW = 32
EBLK = 128
W0_BLKS = 68
W1_BLKS = 16
TOT_BLKS = 16 * (W0_BLKS + W1_BLKS)
E_PAD = EBLK * TOT_BLKS
WMAX = max(W0_BLKS, W1_BLKS)
RPT = N_PAD // 16


def _sc_edge_body(h_hbm, asrc_hbm, adst_hbm, src_hbm, dst_hbm, z2_hbm, z1_hbm,
                  u_out, d_out,
                  asrc_v, adst_v, srcall, dstall,
                  dstb0, dstb1, s0, s1, rows0, rows1, u_sh, d_sh,
                  gsem0, gsem1, usem0, usem1, dsem0, dsem1):
    cid = lax.axis_index("c")
    sid = lax.axis_index("s")

    pltpu.sync_copy(z2_hbm.at[pl.ds(sid * RPT, RPT)],
                    u_sh.at[pl.ds(sid * RPT, RPT)])
    pltpu.sync_copy(z1_hbm.at[pl.ds(sid * RPT, RPT)],
                    d_sh.at[pl.ds(sid * RPT, RPT)])
    pltpu.sync_copy(asrc_hbm, asrc_v)
    pltpu.sync_copy(adst_hbm, adst_v)
    nb = jnp.where(cid == 0, W0_BLKS, W1_BLKS)
    base_blk = jnp.where(cid == 0, sid * W0_BLKS,
                         16 * W0_BLKS + sid * W1_BLKS)
    base_e = base_blk * EBLK

    @pl.when(cid == 0)
    def _stage0():
        pltpu.sync_copy(src_hbm.at[pl.ds(base_e, W0_BLKS * EBLK)],
                        srcall.at[pl.ds(0, W0_BLKS * EBLK)])
        pltpu.sync_copy(dst_hbm.at[pl.ds(base_e, W0_BLKS * EBLK)],
                        dstall.at[pl.ds(0, W0_BLKS * EBLK)])

    @pl.when(cid == 1)
    def _stage1():
        pltpu.sync_copy(src_hbm.at[pl.ds(base_e, W1_BLKS * EBLK)],
                        srcall.at[pl.ds(0, W1_BLKS * EBLK)])
        pltpu.sync_copy(dst_hbm.at[pl.ds(base_e, W1_BLKS * EBLK)],
                        dstall.at[pl.ds(0, W1_BLKS * EBLK)])

    plsc.subcore_barrier()

    bufs = ((dstb0, s0, rows0, gsem0, usem0, dsem0),
            (dstb1, s1, rows1, gsem1, usem1, dsem1))
    NB = nb

    def _fill_and_gather(buf, b):
        dstb, _, rows_v, gsem, _, _ = buf
        ebl = b * EBLK
        for i in range(EBLK // 16):
            dstb[pl.ds(i * 16, 16)] = dstall[pl.ds(ebl + i * 16, 16)]
        pltpu.async_copy(h_hbm.at[srcall.at[pl.ds(ebl, EBLK)]], rows_v, gsem)

    _fill_and_gather(bufs[0], 0)

    def pair_body(pb, carry):
        for p in range(2):
            dstb, s_v, rows_v, gsem, usem, dsem = bufs[p]
            q = 1 - p
            dstb_q, s_q, rows_q, gsem_q, usem_q, dsem_q = bufs[q]
            b = pb * 2 + p
            ebl = b * EBLK

            @pl.when(b + 1 < NB)
            def _prefetch():
                @pl.when(b >= 1)
                def _drain():
                    pltpu.make_async_copy(rows_q, u_sh.at[dstb_q], usem_q).wait()
                    pltpu.make_async_copy(s_q, d_sh.at[dstb_q], dsem_q).wait()
                _fill_and_gather(bufs[q], b + 1)

            svals = []
            for i in range(EBLK // 16):
                idx_s = srcall[pl.ds(ebl + i * 16, 16)]
                idx_d = dstb[pl.ds(i * 16, 16)]
                e = (plsc.load_gather(asrc_v, [idx_s])
                     + plsc.load_gather(adst_v, [idx_d]))
                e = jnp.where(e >= 0.0, e, 0.5 * e)
                sv = jnp.exp(e)
                s_v[pl.ds(i * 16, 16)] = sv
                svals.append(sv)

            pltpu.make_async_copy(h_hbm.at[srcall.at[pl.ds(ebl, EBLK)]],
                                  rows_v, gsem).wait()

            for i in range(EBLK // 16):
                sv = svals[i]
                for l in range(16):
                    sk = sv[l]
                    k = i * 16 + l
                    for c in range(OUT_DIM // 16):
                        rows_v[k, pl.ds(c * 16, 16)] = (
                            rows_v[k, pl.ds(c * 16, 16)] * sk)

            pltpu.async_copy(rows_v, u_sh.at[dstb], usem, add=True)
            pltpu.async_copy(s_v, d_sh.at[dstb], dsem, add=True)
        return carry

    lax.fori_loop(0, nb // 2, pair_body, 0)
    for p in range(2):
        dstb, s_v, rows_v, gsem, usem, dsem = bufs[p]
        pltpu.make_async_copy(rows_v, u_sh.at[dstb], usem).wait()
        pltpu.make_async_copy(s_v, d_sh.at[dstb], dsem).wait()
    plsc.subcore_barrier()

    pltpu.sync_copy(u_sh.at[pl.ds(sid * RPT, RPT)],
                    u_out.at[cid, pl.ds(sid * RPT, RPT)])
    pltpu.sync_copy(d_sh.at[pl.ds(sid * RPT, RPT)],
                    d_out.at[cid, pl.ds(sid * RPT, RPT)])


_sc_edge = functools.partial(
    pl.kernel,
    out_type=(
        jax.ShapeDtypeStruct((2, N_PAD, OUT_DIM), jnp.float32),
        jax.ShapeDtypeStruct((2, N_PAD), jnp.float32),
    ),
    mesh=plsc.VectorSubcoreMesh(core_axis_name="c", subcore_axis_name="s"),
    compiler_params=pltpu.CompilerParams(needs_layout_passes=False,
                                         use_tc_tiling_on_sc=False),
    scratch_types=[
        pltpu.VMEM((N_PAD,), jnp.float32),
        pltpu.VMEM((N_PAD,), jnp.float32),
        pltpu.VMEM((WMAX * EBLK,), jnp.int32),
        pltpu.VMEM((WMAX * EBLK,), jnp.int32),
        pltpu.VMEM((EBLK,), jnp.int32),
        pltpu.VMEM((EBLK,), jnp.int32),
        pltpu.VMEM((EBLK,), jnp.float32),
        pltpu.VMEM((EBLK,), jnp.float32),
        pltpu.VMEM((EBLK, OUT_DIM), jnp.float32),
        pltpu.VMEM((EBLK, OUT_DIM), jnp.float32),
        pltpu.VMEM_SHARED((N_PAD, OUT_DIM), jnp.float32),
        pltpu.VMEM_SHARED((N_PAD,), jnp.float32),
        pltpu.SemaphoreType.DMA,
        pltpu.SemaphoreType.DMA,
        pltpu.SemaphoreType.DMA,
        pltpu.SemaphoreType.DMA,
        pltpu.SemaphoreType.DMA,
        pltpu.SemaphoreType.DMA,
    ],
)(_sc_edge_body)


def _proj_body(x_ref, w_ref, h_ref, as_ref, ad_ref):
    i = pl.program_id(0)
    blk = h_ref.shape[0]
    hx = jnp.dot(x_ref[...], w_ref[...], preferred_element_type=jnp.float32)
    rows = i * blk + lax.broadcasted_iota(jnp.int32, hx.shape, 0)
    hx = jnp.where(rows < N, hx, 0.0)
    h_ref[...] = hx[:, :OUT_DIM]
    as_ref[...] = hx[:, OUT_DIM:OUT_DIM + 1]
    ad_ref[...] = hx[:, OUT_DIM + 1:OUT_DIM + 2]


def _embed_body(u0, u1, d0, d1, b_ref, o_ref):
    den = d0[...] + d1[...] + 1e-16
    e = (u0[...] + u1[...]) / den + b_ref[...]
    o_ref[...] = jnp.where(e >= 0.0, e, 0.01 * e)


def _decoder_body(a_ref, b_ref, o_ref):
    acc = lax.dot_general(a_ref[...], b_ref[...],
                          (((1,), (1,)), ((), ())),
                          preferred_element_type=jnp.float32)
    o_ref[...] = jax.nn.sigmoid(acc)


def kernel(x, edge_index, W, att_src, att_dst, bias):
    f32 = jnp.float32
    v_src = att_src @ W
    v_dst = att_dst @ W
    w_ext = jnp.concatenate(
        [W, v_src[None, :], v_dst[None, :],
         jnp.zeros((IN_DIM - OUT_DIM - 2, IN_DIM), f32)], axis=0)

    BM = 512
    h_pad, asrc_p, adst_p = pl.pallas_call(
        _proj_body,
        grid=(N_PAD // BM,),
        in_specs=[pl.BlockSpec((BM, IN_DIM), lambda i: (i, 0)),
                  pl.BlockSpec((IN_DIM, IN_DIM), lambda i: (0, 0))],
        out_specs=[pl.BlockSpec((BM, OUT_DIM), lambda i: (i, 0)),
                   pl.BlockSpec((BM, 1), lambda i: (i, 0)),
                   pl.BlockSpec((BM, 1), lambda i: (i, 0))],
        out_shape=[jax.ShapeDtypeStruct((N_PAD, OUT_DIM), f32),
                   jax.ShapeDtypeStruct((N_PAD, 1), f32),
                   jax.ShapeDtypeStruct((N_PAD, 1), f32)],
    )(x, w_ext.T)
    asrc_pad = asrc_p.reshape(N_PAD)
    adst_pad = adst_p.reshape(N_PAD)

    loops = jnp.arange(N, dtype=jnp.int32)
    padv = jnp.full((E_PAD - edge_index.shape[1] - N,), N, jnp.int32)
    src = jnp.concatenate([edge_index[0].astype(jnp.int32), loops, padv])
    dst = jnp.concatenate([edge_index[1].astype(jnp.int32), loops, padv])

    u, d = _sc_edge(h_pad, asrc_pad, adst_pad, src, dst,
                    jnp.zeros((N_PAD, OUT_DIM), f32),
                    jnp.zeros((N_PAD,), f32))

    d0 = d[0].reshape(N_PAD, 1)
    d1 = d[1].reshape(N_PAD, 1)
    embed = pl.pallas_call(
        _embed_body,
        grid=(pl.cdiv(N, BM),),
        in_specs=[pl.BlockSpec((BM, OUT_DIM), lambda i: (i, 0)),
                  pl.BlockSpec((BM, OUT_DIM), lambda i: (i, 0)),
                  pl.BlockSpec((BM, 1), lambda i: (i, 0)),
                  pl.BlockSpec((BM, 1), lambda i: (i, 0)),
                  pl.BlockSpec((1, OUT_DIM), lambda i: (0, 0))],
        out_specs=pl.BlockSpec((BM, OUT_DIM), lambda i: (i, 0)),
        out_shape=jax.ShapeDtypeStruct((N, OUT_DIM), f32),
    )(u[0], u[1], d0, d1, bias.reshape(1, OUT_DIM))

    BR, BC = 512, 10240
    xr = pl.pallas_call(
        _decoder_body,
        grid=(pl.cdiv(N, BR), pl.cdiv(N, BC)),
        in_specs=[pl.BlockSpec((BR, OUT_DIM), lambda i, j: (i, 0)),
                  pl.BlockSpec((BC, OUT_DIM), lambda i, j: (j, 0))],
        out_specs=pl.BlockSpec((BR, BC), lambda i, j: (i, j)),
        out_shape=jax.ShapeDtypeStruct((N, N), f32),
    )(embed, embed)

    return (xr, embed)


# speedup vs baseline: 1.0233x; 1.0233x over previous
import functools

import jax
import jax.numpy as jnp
from jax import lax
from jax.experimental import pallas as pl
from jax.experimental.pallas import tpu as pltpu
from jax.experimental.pallas import tpu_sc as plsc

N = 10000
IN_DIM = 128
OUT_DIM = 64
N_PAD = 10240
NW = 32
EBLK = 128
W0_BLKS = 62
W1_BLKS = 22
TOT_BLKS = 16 * (W0_BLKS + W1_BLKS)
E_PAD = EBLK * TOT_BLKS
WMAX = max(W0_BLKS, W1_BLKS)
RPT = N_PAD // 16


def _sc_edge_body(h_hbm, asrc_hbm, adst_hbm, src_hbm, dst_hbm, z2_hbm, z1_hbm,
                  u_out, d_out,
                  asrc_v, adst_v, srcall, dstall,
                  dstb0, dstb1, s0, s1, rows0, rows1, u_sh, d_sh,
                  gsem0, gsem1, usem0, usem1, dsem0, dsem1):
    cid = lax.axis_index("c")
    sid = lax.axis_index("s")

    pltpu.sync_copy(z2_hbm.at[pl.ds(sid * RPT, RPT)],
                    u_sh.at[pl.ds(sid * RPT, RPT)])
    pltpu.sync_copy(z1_hbm.at[pl.ds(sid * RPT, RPT)],
                    d_sh.at[pl.ds(sid * RPT, RPT)])
    pltpu.sync_copy(asrc_hbm, asrc_v)
    pltpu.sync_copy(adst_hbm, adst_v)
    nb = jnp.where(cid == 0, W0_BLKS, W1_BLKS)
    base_blk = jnp.where(cid == 0, sid * W0_BLKS,
                         16 * W0_BLKS + sid * W1_BLKS)
    base_e = base_blk * EBLK

    @pl.when(cid == 0)
    def _stage0():
        pltpu.sync_copy(src_hbm.at[pl.ds(base_e, W0_BLKS * EBLK)],
                        srcall.at[pl.ds(0, W0_BLKS * EBLK)])
        pltpu.sync_copy(dst_hbm.at[pl.ds(base_e, W0_BLKS * EBLK)],
                        dstall.at[pl.ds(0, W0_BLKS * EBLK)])

    @pl.when(cid == 1)
    def _stage1():
        pltpu.sync_copy(src_hbm.at[pl.ds(base_e, W1_BLKS * EBLK)],
                        srcall.at[pl.ds(0, W1_BLKS * EBLK)])
        pltpu.sync_copy(dst_hbm.at[pl.ds(base_e, W1_BLKS * EBLK)],
                        dstall.at[pl.ds(0, W1_BLKS * EBLK)])

    plsc.subcore_barrier()

    bufs = ((dstb0, s0, rows0, gsem0, usem0, dsem0),
            (dstb1, s1, rows1, gsem1, usem1, dsem1))
    NB = nb

    def _fill_and_gather(buf, b):
        dstb, _, rows_v, gsem, _, _ = buf
        ebl = b * EBLK
        for i in range(EBLK // 16):
            dstb[pl.ds(i * 16, 16)] = dstall[pl.ds(ebl + i * 16, 16)]
        pltpu.async_copy(h_hbm.at[srcall.at[pl.ds(ebl, EBLK)]], rows_v, gsem)

    _fill_and_gather(bufs[0], 0)

    def pair_body(pb, carry):
        for p in range(2):
            dstb, s_v, rows_v, gsem, usem, dsem = bufs[p]
            q = 1 - p
            dstb_q, s_q, rows_q, gsem_q, usem_q, dsem_q = bufs[q]
            b = pb * 2 + p
            ebl = b * EBLK

            @pl.when(b + 1 < NB)
            def _prefetch():
                @pl.when(b >= 1)
                def _drain():
                    pltpu.make_async_copy(rows_q, u_sh.at[dstb_q], usem_q).wait()
                    pltpu.make_async_copy(s_q, d_sh.at[dstb_q], dsem_q).wait()
                _fill_and_gather(bufs[q], b + 1)

            svals = []
            for i in range(EBLK // 16):
                idx_s = srcall[pl.ds(ebl + i * 16, 16)]
                idx_d = dstb[pl.ds(i * 16, 16)]
                e = (plsc.load_gather(asrc_v, [idx_s])
                     + plsc.load_gather(adst_v, [idx_d]))
                e = jnp.where(e >= 0.0, e, 0.5 * e)
                sv = jnp.exp(e)
                s_v[pl.ds(i * 16, 16)] = sv
                svals.append(sv)

            pltpu.make_async_copy(h_hbm.at[srcall.at[pl.ds(ebl, EBLK)]],
                                  rows_v, gsem).wait()

            for i in range(EBLK // 16):
                sv = svals[i]
                for l in range(16):
                    sk = sv[l]
                    k = i * 16 + l
                    for c in range(OUT_DIM // 16):
                        rows_v[k, pl.ds(c * 16, 16)] = (
                            rows_v[k, pl.ds(c * 16, 16)] * sk)

            pltpu.async_copy(rows_v, u_sh.at[dstb], usem, add=True)
            pltpu.async_copy(s_v, d_sh.at[dstb], dsem, add=True)
        return carry

    lax.fori_loop(0, nb // 2, pair_body, 0)
    for p in range(2):
        dstb, s_v, rows_v, gsem, usem, dsem = bufs[p]
        pltpu.make_async_copy(rows_v, u_sh.at[dstb], usem).wait()
        pltpu.make_async_copy(s_v, d_sh.at[dstb], dsem).wait()
    plsc.subcore_barrier()

    pltpu.sync_copy(u_sh.at[pl.ds(sid * RPT, RPT)],
                    u_out.at[cid, pl.ds(sid * RPT, RPT)])
    pltpu.sync_copy(d_sh.at[pl.ds(sid * RPT, RPT)],
                    d_out.at[cid, pl.ds(sid * RPT, RPT)])


_sc_edge = functools.partial(
    pl.kernel,
    out_type=(
        jax.ShapeDtypeStruct((2, N_PAD, OUT_DIM), jnp.float32),
        jax.ShapeDtypeStruct((2, N_PAD), jnp.float32),
    ),
    mesh=plsc.VectorSubcoreMesh(core_axis_name="c", subcore_axis_name="s"),
    compiler_params=pltpu.CompilerParams(needs_layout_passes=False,
                                         use_tc_tiling_on_sc=False),
    scratch_types=[
        pltpu.VMEM((N_PAD,), jnp.float32),
        pltpu.VMEM((N_PAD,), jnp.float32),
        pltpu.VMEM((WMAX * EBLK,), jnp.int32),
        pltpu.VMEM((WMAX * EBLK,), jnp.int32),
        pltpu.VMEM((EBLK,), jnp.int32),
        pltpu.VMEM((EBLK,), jnp.int32),
        pltpu.VMEM((EBLK,), jnp.float32),
        pltpu.VMEM((EBLK,), jnp.float32),
        pltpu.VMEM((EBLK, OUT_DIM), jnp.float32),
        pltpu.VMEM((EBLK, OUT_DIM), jnp.float32),
        pltpu.VMEM_SHARED((N_PAD, OUT_DIM), jnp.float32),
        pltpu.VMEM_SHARED((N_PAD,), jnp.float32),
        pltpu.SemaphoreType.DMA,
        pltpu.SemaphoreType.DMA,
        pltpu.SemaphoreType.DMA,
        pltpu.SemaphoreType.DMA,
        pltpu.SemaphoreType.DMA,
        pltpu.SemaphoreType.DMA,
    ],
)(_sc_edge_body)


def _proj_body(x_ref, w_ref, h_ref, as_ref, ad_ref):
    i = pl.program_id(0)
    blk = h_ref.shape[0]
    hx = jnp.dot(x_ref[...], w_ref[...], preferred_element_type=jnp.float32)
    rows = i * blk + lax.broadcasted_iota(jnp.int32, hx.shape, 0)
    hx = jnp.where(rows < N, hx, 0.0)
    h_ref[...] = hx[:, :OUT_DIM]
    as_ref[...] = hx[:, OUT_DIM:OUT_DIM + 1]
    ad_ref[...] = hx[:, OUT_DIM + 1:OUT_DIM + 2]


def _dec_body(u0, u1, d0, d1, b_ref, xr_ref, emb_ref, emb_s):
    i = pl.program_id(0)
    br = emb_ref.shape[0]

    @pl.when(i == 0)
    def _compute_embed():
        den = d0[...] + d1[...] + 1e-16
        e = (u0[...] + u1[...]) / den + b_ref[...]
        emb_s[...] = jnp.where(e >= 0.0, e, 0.01 * e)

    a = emb_s[pl.ds(i * br, br), :]
    acc = lax.dot_general(a, emb_s[...], (((1,), (1,)), ((), ())),
                          preferred_element_type=jnp.float32)
    xr_ref[...] = jax.nn.sigmoid(acc)
    emb_ref[...] = a


def kernel(x, edge_index, W, att_src, att_dst, bias):
    f32 = jnp.float32
    v_src = att_src @ W
    v_dst = att_dst @ W
    w_ext = jnp.concatenate(
        [W, v_src[None, :], v_dst[None, :],
         jnp.zeros((IN_DIM - OUT_DIM - 2, IN_DIM), f32)], axis=0)

    BM = 512
    h_pad, asrc_p, adst_p = pl.pallas_call(
        _proj_body,
        grid=(N_PAD // BM,),
        in_specs=[pl.BlockSpec((BM, IN_DIM), lambda i: (i, 0)),
                  pl.BlockSpec((IN_DIM, IN_DIM), lambda i: (0, 0))],
        out_specs=[pl.BlockSpec((BM, OUT_DIM), lambda i: (i, 0)),
                   pl.BlockSpec((BM, 1), lambda i: (i, 0)),
                   pl.BlockSpec((BM, 1), lambda i: (i, 0))],
        out_shape=[jax.ShapeDtypeStruct((N_PAD, OUT_DIM), f32),
                   jax.ShapeDtypeStruct((N_PAD, 1), f32),
                   jax.ShapeDtypeStruct((N_PAD, 1), f32)],
    )(x, w_ext.T)
    asrc_pad = asrc_p.reshape(N_PAD)
    adst_pad = adst_p.reshape(N_PAD)

    loops = jnp.arange(N, dtype=jnp.int32)
    padv = jnp.full((E_PAD - edge_index.shape[1] - N,), N, jnp.int32)
    src = jnp.concatenate([edge_index[0].astype(jnp.int32), loops, padv])
    dst = jnp.concatenate([edge_index[1].astype(jnp.int32), loops, padv])

    u, d = _sc_edge(h_pad, asrc_pad, adst_pad, src, dst,
                    jnp.zeros((N_PAD, OUT_DIM), f32),
                    jnp.zeros((N_PAD,), f32))

    d0 = d[0].reshape(N_PAD, 1)
    d1 = d[1].reshape(N_PAD, 1)
    BR = 256
    xr, embed = pl.pallas_call(
        _dec_body,
        grid=(pl.cdiv(N, BR),),
        in_specs=[pl.BlockSpec((N_PAD, OUT_DIM), lambda i: (0, 0)),
                  pl.BlockSpec((N_PAD, OUT_DIM), lambda i: (0, 0)),
                  pl.BlockSpec((N_PAD, 1), lambda i: (0, 0)),
                  pl.BlockSpec((N_PAD, 1), lambda i: (0, 0)),
                  pl.BlockSpec((1, OUT_DIM), lambda i: (0, 0))],
        out_specs=[pl.BlockSpec((BR, N_PAD), lambda i: (i, 0)),
                   pl.BlockSpec((BR, OUT_DIM), lambda i: (i, 0))],
        out_shape=[jax.ShapeDtypeStruct((N, N), f32),
                   jax.ShapeDtypeStruct((N, OUT_DIM), f32)],
        scratch_shapes=[pltpu.VMEM((N_PAD, OUT_DIM), f32)],
    )(u[0], u[1], d0, d1, bias.reshape(1, OUT_DIM))

    return (xr, embed)


# speedup vs baseline: 1.0413x; 1.0176x over previous
import functools

import jax
import jax.numpy as jnp
from jax import lax
from jax.experimental import pallas as pl
from jax.experimental.pallas import tpu as pltpu
from jax.experimental.pallas import tpu_sc as plsc

N = 10000
IN_DIM = 128
OUT_DIM = 64
N_PAD = 10240
NW = 32
EBLK = 128
W0_BLKS = 62
W1_BLKS = 22
TOT_BLKS = 16 * (W0_BLKS + W1_BLKS)
E_PAD = EBLK * TOT_BLKS
WMAX = max(W0_BLKS, W1_BLKS)
RPT = N_PAD // 16


def _sc_edge_body(h_hbm, asrc_hbm, adst_hbm, src_hbm, dst_hbm, z2_hbm, z1_hbm,
                  u_out, d_out,
                  asrc_v, adst_v, srcall, dstall,
                  dstb0, dstb1, s0, s1, rows0, rows1, u_sh, d_sh,
                  gsem0, gsem1, usem0, usem1, dsem0, dsem1):
    cid = lax.axis_index("c")
    sid = lax.axis_index("s")

    pltpu.sync_copy(z2_hbm.at[pl.ds(sid * RPT, RPT)],
                    u_sh.at[pl.ds(sid * RPT, RPT)])
    pltpu.sync_copy(z1_hbm.at[pl.ds(sid * RPT, RPT)],
                    d_sh.at[pl.ds(sid * RPT, RPT)])
    pltpu.sync_copy(asrc_hbm, asrc_v)
    pltpu.sync_copy(adst_hbm, adst_v)
    nb = jnp.where(cid == 0, W0_BLKS, W1_BLKS)
    base_blk = jnp.where(cid == 0, sid * W0_BLKS,
                         16 * W0_BLKS + sid * W1_BLKS)
    base_e = base_blk * EBLK

    @pl.when(cid == 0)
    def _stage0():
        pltpu.sync_copy(src_hbm.at[pl.ds(base_e, W0_BLKS * EBLK)],
                        srcall.at[pl.ds(0, W0_BLKS * EBLK)])
        pltpu.sync_copy(dst_hbm.at[pl.ds(base_e, W0_BLKS * EBLK)],
                        dstall.at[pl.ds(0, W0_BLKS * EBLK)])

    @pl.when(cid == 1)
    def _stage1():
        pltpu.sync_copy(src_hbm.at[pl.ds(base_e, W1_BLKS * EBLK)],
                        srcall.at[pl.ds(0, W1_BLKS * EBLK)])
        pltpu.sync_copy(dst_hbm.at[pl.ds(base_e, W1_BLKS * EBLK)],
                        dstall.at[pl.ds(0, W1_BLKS * EBLK)])

    plsc.subcore_barrier()

    bufs = ((dstb0, s0, rows0, gsem0, usem0, dsem0),
            (dstb1, s1, rows1, gsem1, usem1, dsem1))
    NB = nb

    def _fill_and_gather(buf, b):
        dstb, _, rows_v, gsem, _, _ = buf
        ebl = b * EBLK
        for i in range(EBLK // 16):
            dstb[pl.ds(i * 16, 16)] = dstall[pl.ds(ebl + i * 16, 16)]
        pltpu.async_copy(h_hbm.at[srcall.at[pl.ds(ebl, EBLK)]], rows_v, gsem)

    _fill_and_gather(bufs[0], 0)

    def pair_body(pb, carry):
        for p in range(2):
            dstb, s_v, rows_v, gsem, usem, dsem = bufs[p]
            q = 1 - p
            dstb_q, s_q, rows_q, gsem_q, usem_q, dsem_q = bufs[q]
            b = pb * 2 + p
            ebl = b * EBLK

            @pl.when(b + 1 < NB)
            def _prefetch():
                @pl.when(b >= 1)
                def _drain():
                    pltpu.make_async_copy(rows_q, u_sh.at[dstb_q], usem_q).wait()
                    pltpu.make_async_copy(s_q, d_sh.at[dstb_q], dsem_q).wait()
                _fill_and_gather(bufs[q], b + 1)

            svals = []
            for i in range(EBLK // 16):
                idx_s = srcall[pl.ds(ebl + i * 16, 16)]
                idx_d = dstb[pl.ds(i * 16, 16)]
                e = (plsc.load_gather(asrc_v, [idx_s])
                     + plsc.load_gather(adst_v, [idx_d]))
                e = jnp.where(e >= 0.0, e, 0.5 * e)
                sv = jnp.exp(e)
                s_v[pl.ds(i * 16, 16)] = sv
                svals.append(sv)

            pltpu.make_async_copy(h_hbm.at[srcall.at[pl.ds(ebl, EBLK)]],
                                  rows_v, gsem).wait()

            for i in range(EBLK // 16):
                sv = svals[i]
                for l in range(16):
                    sk = sv[l]
                    k = i * 16 + l
                    for c in range(OUT_DIM // 16):
                        rows_v[k, pl.ds(c * 16, 16)] = (
                            rows_v[k, pl.ds(c * 16, 16)] * sk)

            pltpu.async_copy(rows_v, u_sh.at[dstb], usem, add=True)
            pltpu.async_copy(s_v, d_sh.at[dstb], dsem, add=True)
        return carry

    lax.fori_loop(0, nb // 2, pair_body, 0)
    for p in range(2):
        dstb, s_v, rows_v, gsem, usem, dsem = bufs[p]
        pltpu.make_async_copy(rows_v, u_sh.at[dstb], usem).wait()
        pltpu.make_async_copy(s_v, d_sh.at[dstb], dsem).wait()
    plsc.subcore_barrier()

    pltpu.sync_copy(u_sh.at[pl.ds(sid * RPT, RPT)],
                    u_out.at[cid, pl.ds(sid * RPT, RPT)])
    pltpu.sync_copy(d_sh.at[pl.ds(sid * RPT, RPT)],
                    d_out.at[cid, pl.ds(sid * RPT, RPT)])


_sc_edge = functools.partial(
    pl.kernel,
    out_type=(
        jax.ShapeDtypeStruct((2, N_PAD, OUT_DIM), jnp.float32),
        jax.ShapeDtypeStruct((2, N_PAD), jnp.float32),
    ),
    mesh=plsc.VectorSubcoreMesh(core_axis_name="c", subcore_axis_name="s"),
    compiler_params=pltpu.CompilerParams(needs_layout_passes=False,
                                         use_tc_tiling_on_sc=False),
    scratch_types=[
        pltpu.VMEM((N_PAD,), jnp.float32),
        pltpu.VMEM((N_PAD,), jnp.float32),
        pltpu.VMEM((WMAX * EBLK,), jnp.int32),
        pltpu.VMEM((WMAX * EBLK,), jnp.int32),
        pltpu.VMEM((EBLK,), jnp.int32),
        pltpu.VMEM((EBLK,), jnp.int32),
        pltpu.VMEM((EBLK,), jnp.float32),
        pltpu.VMEM((EBLK,), jnp.float32),
        pltpu.VMEM((EBLK, OUT_DIM), jnp.float32),
        pltpu.VMEM((EBLK, OUT_DIM), jnp.float32),
        pltpu.VMEM_SHARED((N_PAD, OUT_DIM), jnp.float32),
        pltpu.VMEM_SHARED((N_PAD,), jnp.float32),
        pltpu.SemaphoreType.DMA,
        pltpu.SemaphoreType.DMA,
        pltpu.SemaphoreType.DMA,
        pltpu.SemaphoreType.DMA,
        pltpu.SemaphoreType.DMA,
        pltpu.SemaphoreType.DMA,
    ],
)(_sc_edge_body)


def _proj_body(x_ref, w_ref, h_ref, as_ref, ad_ref):
    i = pl.program_id(0)
    blk = h_ref.shape[0]
    hx = jnp.dot(x_ref[...], w_ref[...], preferred_element_type=jnp.float32)
    rows = i * blk + lax.broadcasted_iota(jnp.int32, hx.shape, 0)
    hx = jnp.where(rows < N, hx, 0.0)
    h_ref[...] = hx[:, :OUT_DIM]
    as_ref[...] = hx[:, OUT_DIM:OUT_DIM + 1]
    ad_ref[...] = hx[:, OUT_DIM + 1:OUT_DIM + 2]


def _dec_body(u0, u1, d0, d1, b_ref, xr_ref, emb_ref, emb_s):
    i = pl.program_id(0)
    br = emb_ref.shape[0]

    @pl.when(i == 0)
    def _compute_embed():
        den = d0[...] + d1[...] + 1e-16
        e = (u0[...] + u1[...]) / den + b_ref[...]
        emb_s[...] = jnp.where(e >= 0.0, e, 0.01 * e)

    a = emb_s[pl.ds(i * br, br), :]
    acc = lax.dot_general(a, emb_s[...], (((1,), (1,)), ((), ())),
                          preferred_element_type=jnp.float32)
    xr_ref[...] = jax.nn.sigmoid(acc)
    emb_ref[...] = a


def kernel(x, edge_index, W, att_src, att_dst, bias):
    f32 = jnp.float32
    v_src = att_src @ W
    v_dst = att_dst @ W
    w_ext = jnp.concatenate(
        [W, v_src[None, :], v_dst[None, :],
         jnp.zeros((IN_DIM - OUT_DIM - 2, IN_DIM), f32)], axis=0)

    BM = 512
    h_pad, asrc_p, adst_p = pl.pallas_call(
        _proj_body,
        grid=(N_PAD // BM,),
        in_specs=[pl.BlockSpec((BM, IN_DIM), lambda i: (i, 0)),
                  pl.BlockSpec((IN_DIM, IN_DIM), lambda i: (0, 0))],
        out_specs=[pl.BlockSpec((BM, OUT_DIM), lambda i: (i, 0)),
                   pl.BlockSpec((BM, 1), lambda i: (i, 0)),
                   pl.BlockSpec((BM, 1), lambda i: (i, 0))],
        out_shape=[jax.ShapeDtypeStruct((N_PAD, OUT_DIM), f32),
                   jax.ShapeDtypeStruct((N_PAD, 1), f32),
                   jax.ShapeDtypeStruct((N_PAD, 1), f32)],
    )(x, w_ext.T)
    asrc_pad = asrc_p.reshape(N_PAD)
    adst_pad = adst_p.reshape(N_PAD)

    loops = jnp.arange(N, dtype=jnp.int32)
    padv = jnp.full((E_PAD - edge_index.shape[1] - N,), N, jnp.int32)
    src = jnp.concatenate([edge_index[0].astype(jnp.int32), loops, padv])
    dst = jnp.concatenate([edge_index[1].astype(jnp.int32), loops, padv])

    u, d = _sc_edge(h_pad, asrc_pad, adst_pad, src, dst,
                    jnp.zeros((N_PAD, OUT_DIM), f32),
                    jnp.zeros((N_PAD,), f32))

    d0 = d[0].reshape(N_PAD, 1)
    d1 = d[1].reshape(N_PAD, 1)
    BR = 384
    xr, embed = pl.pallas_call(
        _dec_body,
        grid=(pl.cdiv(N, BR),),
        in_specs=[pl.BlockSpec((N_PAD, OUT_DIM), lambda i: (0, 0)),
                  pl.BlockSpec((N_PAD, OUT_DIM), lambda i: (0, 0)),
                  pl.BlockSpec((N_PAD, 1), lambda i: (0, 0)),
                  pl.BlockSpec((N_PAD, 1), lambda i: (0, 0)),
                  pl.BlockSpec((1, OUT_DIM), lambda i: (0, 0))],
        out_specs=[pl.BlockSpec((BR, N_PAD), lambda i: (i, 0)),
                   pl.BlockSpec((BR, OUT_DIM), lambda i: (i, 0))],
        out_shape=[jax.ShapeDtypeStruct((N, N), f32),
                   jax.ShapeDtypeStruct((N, OUT_DIM), f32)],
        scratch_shapes=[pltpu.VMEM((N_PAD, OUT_DIM), f32)],
    )(u[0], u[1], d0, d1, bias.reshape(1, OUT_DIM))

    return (xr, embed)
